# baseline (speedup 1.0000x reference)
import functools
import math

import ml_dtypes
import numpy as np
import jax
import jax.numpy as jnp
from jax import lax
from jax.experimental import pallas as pl
from jax.experimental.pallas import tpu as pltpu, tpu_sc as plsc

_NUM_ANGLE = 100
_NUM_RHO = 100
_COLS_PER_STEP = 1280


@functools.lru_cache(maxsize=None)
def _rho_table(H, W, num_angle, num_rho):
    irho = int(math.sqrt(H * H + W * W) + 1) / float(num_rho)
    itheta = math.pi / num_angle
    angles = np.arange(num_angle, dtype=np.float64) * itheta
    cosv = (np.cos(angles) / irho).astype(np.float32)
    sinv = (np.sin(angles) / irho).astype(np.float32)
    ys, xs = np.meshgrid(np.arange(H), np.arange(W), indexing='ij')
    xx = (xs - W // 2).reshape(-1).astype(np.float32)
    yy = (ys - H // 2).reshape(-1).astype(np.float32)
    r = np.round(xx[None, :] * cosv[:, None] + yy[None, :] * sinv[:, None])
    r = r.astype(np.int32) + num_rho // 2
    r = np.clip(r, 0, num_rho - 1)
    return r


@functools.lru_cache(maxsize=None)
def _vote_matrix(H, W, num_angle, num_rho, cols_pad, dtype=ml_dtypes.bfloat16):
    ridx = _rho_table(H, W, num_angle, num_rho)
    HW = H * W
    flat = ridx + (np.arange(num_angle, dtype=np.int32) * num_rho)[:, None]
    n = np.zeros((HW, cols_pad), dtype=dtype)
    n[np.arange(HW)[None, :], flat] = 1
    return n


_SC_ROWS_PER_ANGLE = 104
_SC_ACC_ROWS = 5248
_SC_TRASH = 5200
_SC_HWPAD = 10240
_SC_PASSES = 5
_SC_TILE_ROWS = 128


@functools.lru_cache(maxsize=None)
def _sc_idx_table(H, W, num_angle, num_rho):
    ridx = _rho_table(H, W, num_angle, num_rho)
    apc = num_angle // 2
    idx = np.full((2, apc, _SC_HWPAD), _SC_TRASH, np.int32)
    for c in (0, 1):
        idx[c, :, :H * W] = (np.arange(apc, dtype=np.int32)[:, None]
                             * _SC_ROWS_PER_ANGLE) + ridx[c * apc:(c + 1) * apc]
    idx = idx.reshape(2, apc, _SC_PASSES, 16, _SC_TILE_ROWS)
    return np.ascontiguousarray(idx.transpose(0, 2, 3, 1, 4))


def _sc_call(xt_pad, idx6, zrows):
    mesh = plsc.VectorSubcoreMesh(core_axis_name="c", subcore_axis_name="s")
    apc = _NUM_ANGLE // 2

    @functools.partial(
        pl.kernel, mesh=mesh,
        out_type=jax.ShapeDtypeStruct((2, _SC_ACC_ROWS, 256), jnp.float32),
        scratch_types=[
            pltpu.VMEM((apc, _SC_TILE_ROWS), jnp.int32),
            pltpu.VMEM_SHARED((_SC_ACC_ROWS, 256), jnp.float32),
        ],
    )
    def k(xt_hbm, idx_hbm, z_hbm, out_hbm, idxs, acc):
        c = lax.axis_index("c")
        s = lax.axis_index("s")
        stripe = s * (_SC_ACC_ROWS // 16)
        pltpu.sync_copy(z_hbm.at[pl.ds(0, _SC_ACC_ROWS // 16)],
                        acc.at[pl.ds(stripe, _SC_ACC_ROWS // 16)])
        plsc.subcore_barrier()
        for p in range(_SC_PASSES):
            base = p * (16 * _SC_TILE_ROWS) + s * _SC_TILE_ROWS
            pltpu.sync_copy(idx_hbm.at[c, p, s], idxs)

            def body(a, carry):
                pltpu.sync_copy(xt_hbm.at[pl.ds(base, _SC_TILE_ROWS)],
                                acc.at[idxs.at[a]], add=True)
                return carry

            lax.fori_loop(0, apc, body, 0)
        plsc.subcore_barrier()
        pltpu.sync_copy(acc.at[pl.ds(stripe, _SC_ACC_ROWS // 16)],
                        out_hbm.at[c, pl.ds(stripe, _SC_ACC_ROWS // 16)])

    return k(xt_pad, idx6, zrows)


def _kernel_sc(x):
    B, C, H, W = x.shape
    HW = H * W
    BC = B * C
    apc = _NUM_ANGLE // 2

    xt = x.reshape(BC, HW).T
    xt_pad = jnp.concatenate(
        [xt, jnp.zeros((_SC_HWPAD - HW, BC), jnp.float32)], axis=0)
    idx6 = jnp.asarray(_sc_idx_table(H, W, _NUM_ANGLE, _NUM_RHO))
    zrows = jnp.zeros((_SC_ACC_ROWS // 16, BC), jnp.float32)

    part = _sc_call(xt_pad, idx6, zrows)
    part = part[:, :apc * _SC_ROWS_PER_ANGLE, :]
    part = part.reshape(2, apc, _SC_ROWS_PER_ANGLE, BC)[:, :, :_NUM_RHO, :]
    out = part.transpose(3, 0, 1, 2)
    return out.reshape(B, C, _NUM_ANGLE, _NUM_RHO)


def _dht_body(n_ref, xf_ref, out_ref):
    out_ref[...] = jnp.dot(xf_ref[...], n_ref[...],
                           preferred_element_type=jnp.float32)


def _dht_body_f8(n_ref, xhi_ref, xlo_ref, out_ref):
    n = n_ref[...]
    out_ref[...] = (jnp.dot(xhi_ref[...], n, preferred_element_type=jnp.float32)
                    + jnp.dot(xlo_ref[...], n, preferred_element_type=jnp.float32))


def kernel(x):
    return _kernel_tc_f8(x)


def _kernel_tc(x):
    B, C, H, W = x.shape
    HW = H * W
    BC = B * C
    AR = _NUM_ANGLE * _NUM_RHO
    cols_pad = ((AR + _COLS_PER_STEP - 1) // _COLS_PER_STEP) * _COLS_PER_STEP
    nsteps = cols_pad // _COLS_PER_STEP

    nmat = jnp.asarray(_vote_matrix(H, W, _NUM_ANGLE, _NUM_RHO, cols_pad))
    xf = x.reshape(BC, HW).astype(jnp.bfloat16)

    out = pl.pallas_call(
        _dht_body,
        grid=(nsteps,),
        in_specs=[
            pl.BlockSpec((HW, _COLS_PER_STEP), lambda i: (0, i)),
            pl.BlockSpec((BC, HW), lambda i: (0, 0)),
        ],
        out_specs=pl.BlockSpec((BC, _COLS_PER_STEP), lambda i: (0, i)),
        out_shape=jax.ShapeDtypeStruct((BC, cols_pad), jnp.float32),
    )(nmat, xf)

    return out[:, :AR].reshape(B, C, _NUM_ANGLE, _NUM_RHO)


def _kernel_tc_f8(x):
    B, C, H, W = x.shape
    HW = H * W
    BC = B * C
    AR = _NUM_ANGLE * _NUM_RHO
    cols_pad = ((AR + _COLS_PER_STEP - 1) // _COLS_PER_STEP) * _COLS_PER_STEP
    nsteps = cols_pad // _COLS_PER_STEP

    nmat = jnp.asarray(_vote_matrix(H, W, _NUM_ANGLE, _NUM_RHO, cols_pad,
                                    ml_dtypes.float8_e4m3fn))
    xf = x.reshape(BC, HW)
    xhi = xf.astype(jnp.float8_e4m3fn)
    xlo = (xf - xhi.astype(jnp.float32)).astype(jnp.float8_e4m3fn)

    out = pl.pallas_call(
        _dht_body_f8,
        grid=(nsteps,),
        in_specs=[
            pl.BlockSpec((HW, _COLS_PER_STEP), lambda i: (0, i)),
            pl.BlockSpec((BC, HW), lambda i: (0, 0)),
            pl.BlockSpec((BC, HW), lambda i: (0, 0)),
        ],
        out_specs=pl.BlockSpec((BC, _COLS_PER_STEP), lambda i: (0, i)),
        out_shape=jax.ShapeDtypeStruct((BC, cols_pad), jnp.float32),
    )(nmat, xhi, xlo)

    return out[:, :AR].reshape(B, C, _NUM_ANGLE, _NUM_RHO)


# speedup vs baseline: 1.1303x; 1.1303x over previous
import functools
import math

import ml_dtypes
import numpy as np
import jax
import jax.numpy as jnp
from jax.experimental import pallas as pl

_NUM_ANGLE = 100
_NUM_RHO = 100
_COLS_PER_STEP = 1280


@functools.lru_cache(maxsize=None)
def _rho_table(H, W, num_angle, num_rho):
    irho = int(math.sqrt(H * H + W * W) + 1) / float(num_rho)
    itheta = math.pi / num_angle
    angles = np.arange(num_angle, dtype=np.float64) * itheta
    cosv = (np.cos(angles) / irho).astype(np.float32)
    sinv = (np.sin(angles) / irho).astype(np.float32)
    ys, xs = np.meshgrid(np.arange(H), np.arange(W), indexing='ij')
    xx = (xs - W // 2).reshape(-1).astype(np.float32)
    yy = (ys - H // 2).reshape(-1).astype(np.float32)
    r = np.round(xx[None, :] * cosv[:, None] + yy[None, :] * sinv[:, None])
    r = r.astype(np.int32) + num_rho // 2
    r = np.clip(r, 0, num_rho - 1)
    return r


@functools.lru_cache(maxsize=None)
def _vote_matrix(H, W, num_angle, num_rho, cols_pad):
    ridx = _rho_table(H, W, num_angle, num_rho)
    HW = H * W
    flat = ridx + (np.arange(num_angle, dtype=np.int32) * num_rho)[:, None]
    n = np.zeros((HW, cols_pad), dtype=ml_dtypes.float8_e4m3fn)
    n[np.arange(HW)[None, :], flat] = 1
    return n


def _dht_body(n_ref, xf_ref, out_ref):
    n = n_ref[...].astype(jnp.bfloat16)
    out_ref[...] = jnp.dot(xf_ref[...], n, preferred_element_type=jnp.float32)


def kernel(x):
    B, C, H, W = x.shape
    HW = H * W
    BC = B * C
    AR = _NUM_ANGLE * _NUM_RHO
    cols_pad = ((AR + _COLS_PER_STEP - 1) // _COLS_PER_STEP) * _COLS_PER_STEP
    nsteps = cols_pad // _COLS_PER_STEP

    nmat = jnp.asarray(_vote_matrix(H, W, _NUM_ANGLE, _NUM_RHO, cols_pad))
    xf = x.reshape(BC, HW).astype(jnp.bfloat16)

    out = pl.pallas_call(
        _dht_body,
        grid=(nsteps,),
        in_specs=[
            pl.BlockSpec((HW, _COLS_PER_STEP), lambda i: (0, i)),
            pl.BlockSpec((BC, HW), lambda i: (0, 0)),
        ],
        out_specs=pl.BlockSpec((BC, _COLS_PER_STEP), lambda i: (0, i)),
        out_shape=jax.ShapeDtypeStruct((BC, cols_pad), jnp.float32),
    )(nmat, xf)

    return out[:, :AR].reshape(B, C, _NUM_ANGLE, _NUM_RHO)
